# baseline (speedup 1.0000x reference)
import functools

import jax
import jax.numpy as jnp
from jax import lax
from jax.experimental import pallas as pl
from jax.experimental.pallas import tpu as pltpu
from jax.experimental.pallas import tpu_sc as plsc

_NUM_CLASSES = 1000000
_FEAT = 64
_BATCH = 16384
_LAMBDA = 0.003

_NC = 2
_NS = 16
_NW = _NC * _NS
_ROWS = _BATCH // _NW
_GCHUNK = 128


def _sc_partials(pairs, labels1d, feats_t):
    mesh = plsc.VectorSubcoreMesh(core_axis_name="c", subcore_axis_name="s",
                                  num_cores=_NC, num_subcores=_NS)

    @functools.partial(
        pl.kernel,
        out_type=jax.ShapeDtypeStruct((_NW, 128), jnp.float32),
        mesh=mesh,
        scratch_types=[
            pltpu.VMEM((_ROWS,), jnp.int32),
            pltpu.VMEM((_ROWS,), jnp.int32),
            pltpu.VMEM((_ROWS, 128), jnp.float32),
            pltpu.VMEM((_FEAT, _ROWS), jnp.float32),
            pltpu.VMEM((128,), jnp.float32),
            pltpu.SemaphoreType.DMA,
            pltpu.SemaphoreType.DMA,
        ],
        compiler_params=pltpu.CompilerParams(needs_layout_passes=False),
    )
    def k(p_hbm, labels_hbm, ft_hbm, out_hbm, lbuf, tbuf, rowbuf, fbuf, accv,
          gsem, fsem):
        wid = lax.axis_index("s") * _NC + lax.axis_index("c")
        base = wid * _ROWS

        pltpu.sync_copy(labels_hbm.at[pl.ds(base, _ROWS)], lbuf)
        fcp = pltpu.async_copy(ft_hbm.at[:, pl.ds(base, _ROWS)], fbuf, fsem)

        def idx_body(i, _):
            lv = lbuf[pl.ds(i * 16, 16)]
            tbuf[pl.ds(i * 16, 16)] = lax.shift_right_logical(lv, 1)
            return 0

        lax.fori_loop(0, _ROWS // 16, idx_body, 0)

        gcps = [
            pltpu.async_copy(
                p_hbm.at[tbuf.at[pl.ds(g * _GCHUNK, _GCHUNK)]],
                rowbuf.at[pl.ds(g * _GCHUNK, _GCHUNK)], gsem)
            for g in range(_ROWS // _GCHUNK)
        ]
        fcp.wait()
        for cp in gcps:
            cp.wait()

        lane = lax.iota(jnp.int32, 16)
        one = jnp.full((16,), 1, jnp.int32)

        def comp_body(h, acc):
            off = h * 16
            lv = lbuf[pl.ds(off, 16)]
            half = lax.shift_left(lax.bitwise_and(lv, one), 6)
            slot = lane + off
            for j in range(_FEAT):
                cv = plsc.load_gather(rowbuf, [slot, half + j])
                fv = fbuf[j, pl.ds(off, 16)]
                d = fv - cv
                acc = acc + d * d
            return acc

        acc = lax.fori_loop(0, _ROWS // 16, comp_body,
                            jnp.zeros((16,), jnp.float32))

        zero16 = jnp.zeros((16,), jnp.float32)
        for i in range(8):
            accv[pl.ds(i * 16, 16)] = acc if i == 0 else zero16
        pltpu.sync_copy(accv, out_hbm.at[wid])

    return k(pairs, labels1d, feats_t)


def _reduce_body(p_ref, o_ref):
    s = jnp.sum(p_ref[...]) * (_LAMBDA / (2.0 * _BATCH))
    o_ref[...] = s[None, None]


def kernel(features, labels, centers):
    pairs = centers.reshape(_NUM_CLASSES // 2, 128)
    labels1d = labels.reshape(_BATCH)
    feats_t = features.T
    partials = _sc_partials(pairs, labels1d, feats_t)
    out = pl.pallas_call(
        _reduce_body,
        out_shape=jax.ShapeDtypeStruct((1, 1), jnp.float32),
    )(partials)
    return out[0, 0]


# speedup vs baseline: 1.7087x; 1.7087x over previous
import functools

import jax
import jax.numpy as jnp
from jax import lax
from jax.experimental import pallas as pl
from jax.experimental.pallas import tpu as pltpu
from jax.experimental.pallas import tpu_sc as plsc

_NUM_CLASSES = 1000000
_FEAT = 64
_BATCH = 16384
_LAMBDA = 0.003

_NC = 2
_NS = 16
_NW = _NC * _NS
_ROWS = _BATCH // _NW
_SEG = 256


def _sc_partials(centers, labels1d, features):
    mesh = plsc.VectorSubcoreMesh(core_axis_name="c", subcore_axis_name="s",
                                  num_cores=_NC, num_subcores=_NS)

    @functools.partial(
        pl.kernel,
        out_type=jax.ShapeDtypeStruct((_NW, 128), jnp.float32),
        mesh=mesh,
        scratch_types=[
            pltpu.VMEM((_ROWS,), jnp.int32),
            pltpu.VMEM((_SEG, _FEAT), jnp.float32),
            pltpu.VMEM((_SEG, _FEAT), jnp.float32),
            pltpu.VMEM((128,), jnp.float32),
            [pltpu.SemaphoreType.DMA] * 8,
            pltpu.SemaphoreType.DMA,
        ],
    )
    def k(c_hbm, labels_hbm, f_hbm, out_hbm, lbuf, rowbuf, fbuf, accv, gsems, fsem):
        wid = lax.axis_index("s") * _NC + lax.axis_index("c")
        base = wid * _ROWS

        pltpu.sync_copy(labels_hbm.at[pl.ds(base, _ROWS)], lbuf)

        acc = jnp.zeros((16,), jnp.float32)
        for seg in range(_ROWS // _SEG):
            fcp = pltpu.async_copy(
                f_hbm.at[pl.ds(base + seg * _SEG, _SEG), :], fbuf, fsem)

            def fire_body(g, _, seg=seg):
                lv = lbuf[pl.ds(seg * _SEG + g * 16, 16)]
                for j in range(16):
                    pltpu.async_copy(c_hbm.at[lv[j]], rowbuf.at[g * 16 + j],
                                     gsems[j % 8])
                return 0

            lax.fori_loop(0, _SEG // 16, fire_body, 0)
            fcp.wait()
            for q in range(8):
                pltpu.make_async_copy(
                    f_hbm.at[pl.ds(0, _SEG // 8), :],
                    rowbuf.at[pl.ds(0, _SEG // 8)], gsems[q]).wait()

            def comp_body(r, acc):
                for c in range(_FEAT // 16):
                    fv = fbuf[r, pl.ds(c * 16, 16)]
                    cv = rowbuf[r, pl.ds(c * 16, 16)]
                    d = fv - cv
                    acc = acc + d * d
                return acc

            acc = lax.fori_loop(0, _SEG, comp_body, acc)

        zero16 = jnp.zeros((16,), jnp.float32)
        for i in range(8):
            accv[pl.ds(i * 16, 16)] = acc if i == 0 else zero16
        pltpu.sync_copy(accv, out_hbm.at[wid])

    return k(centers, labels1d, features)


def _reduce_body(p_ref, o_ref):
    s = jnp.sum(p_ref[...]) * (_LAMBDA / (2.0 * _BATCH))
    o_ref[...] = s[None, None]


def kernel(features, labels, centers):
    labels1d = labels.reshape(_BATCH)
    partials = _sc_partials(centers, labels1d, features)
    out = pl.pallas_call(
        _reduce_body,
        out_shape=jax.ShapeDtypeStruct((1, 1), jnp.float32),
    )(partials)
    return out[0, 0]
